# baseline (speedup 1.0000x reference)
import functools

import jax
import jax.numpy as jnp
from jax import lax
from jax.experimental import pallas as pl
from jax.experimental.pallas import tpu as pltpu
from jax.experimental.pallas import tpu_sc as plsc

_EMBED = 128
_NUM_WORKERS = 32
_CHUNK = 128


_NBUF = 4


def _sc_gather(ss, ps, os_idx, ent, rel, offset, length):
    b_per_w = length // _NUM_WORKERS
    rows_t = jax.ShapeDtypeStruct((length, _EMBED), jnp.float32)
    mesh = plsc.VectorSubcoreMesh(core_axis_name="c", subcore_axis_name="s")

    @functools.partial(
        pl.kernel,
        out_type=[rows_t, rows_t, rows_t],
        mesh=mesh,
        scratch_types=(
            [pltpu.VMEM((b_per_w,), jnp.int32) for _ in range(3)]
            + [pltpu.VMEM((_CHUNK, _EMBED), jnp.float32) for _ in range(_NBUF)]
            + [pltpu.SemaphoreType.DMA for _ in range(2 * _NBUF + 1)]
        ),
    )
    def gather_kernel(ss_hbm, ps_hbm, os_hbm, ent_hbm, rel_hbm,
                      s_out, p_out, o_out, sidx, pidx, oidx, *rest):
        bufs = rest[:_NBUF]
        gsem = rest[_NBUF:2 * _NBUF]
        wsem = rest[2 * _NBUF:3 * _NBUF]
        isem = rest[3 * _NBUF]
        wid = lax.axis_index("s") * 2 + lax.axis_index("c")
        base = wid * b_per_w
        gbase = offset + base

        ih = [pltpu.async_copy(src.at[pl.ds(gbase, b_per_w)], dst, isem)
              for src, dst in ((ss_hbm, sidx), (ps_hbm, pidx), (os_hbm, oidx))]
        for h in ih:
            h.wait()

        items = []
        for idxr, tab, out in ((sidx, ent_hbm, s_out),
                               (pidx, rel_hbm, p_out),
                               (oidx, ent_hbm, o_out)):
            for c in range(0, b_per_w, _CHUNK):
                items.append((idxr, c, tab, out))

        g_h = [None] * _NBUF
        w_h = [None] * _NBUF
        prev = None

        def _start_writeback(i, j):
            _, off, _, out = items[i]
            g_h[j].wait()
            w_h[j] = pltpu.async_copy(
                bufs[j], out.at[pl.ds(base + off, _CHUNK)], wsem[j])

        for i, (idxr, off, tab, _) in enumerate(items):
            j = i % _NBUF
            if w_h[j] is not None:
                w_h[j].wait()
                w_h[j] = None
            g_h[j] = pltpu.async_copy(
                tab.at[idxr.at[pl.ds(off, _CHUNK)]], bufs[j], gsem[j])
            if prev is not None:
                _start_writeback(*prev)
            prev = (i, j)
        _start_writeback(*prev)
        for j in range(_NBUF):
            if w_h[j] is not None:
                w_h[j].wait()

    return gather_kernel(ss, ps, os_idx, ent, rel)


def _score_block(s_ref, p_ref, o_ref, out_ref):
    def _norm(x):
        n = jnp.sqrt(jnp.sum(x * x, axis=-1, keepdims=True))
        return x / jnp.maximum(n, 1e-12)

    d = (_norm(o_ref[...]) - _norm(s_ref[...])) - _norm(p_ref[...]) + 1e-6
    out_ref[...] = jnp.sqrt(jnp.sum(d * d, axis=-1))


def _tc_score(s_rows, p_rows, o_rows):
    batch = s_rows.shape[0]
    blk = 2048
    row_spec = pl.BlockSpec((blk, _EMBED), lambda i: (i, 0))
    return pl.pallas_call(
        _score_block,
        grid=(batch // blk,),
        in_specs=[row_spec, row_spec, row_spec],
        out_specs=pl.BlockSpec((blk,), lambda i: (i,)),
        out_shape=jax.ShapeDtypeStruct((batch,), jnp.float32),
    )(s_rows, p_rows, o_rows)


def kernel(ss, ps, os, ent_embedding, rel_embedding):
    ss = ss.astype(jnp.int32)
    ps = ps.astype(jnp.int32)
    os_idx = os.astype(jnp.int32)
    batch = ss.shape[0]
    n_slices = 4
    sl = batch // n_slices
    scores = []
    for k in range(n_slices):
        s_rows, p_rows, o_rows = _sc_gather(
            ss, ps, os_idx, ent_embedding, rel_embedding, k * sl, sl)
        scores.append(_tc_score(s_rows, p_rows, o_rows))
    return jnp.concatenate(scores)


# speedup vs baseline: 1.0396x; 1.0396x over previous
import functools

import jax
import jax.numpy as jnp
from jax import lax
from jax.experimental import pallas as pl
from jax.experimental.pallas import tpu as pltpu
from jax.experimental.pallas import tpu_sc as plsc

_EMBED = 128
_NUM_WORKERS = 32
_CHUNK = 128


_NBUF = 4


def _sc_gather(ss, ps, os_idx, ent, rel, offset, length):
    b_per_w = length // _NUM_WORKERS
    rows_t = jax.ShapeDtypeStruct((length, _EMBED), jnp.float32)
    mesh = plsc.VectorSubcoreMesh(core_axis_name="c", subcore_axis_name="s")

    @functools.partial(
        pl.kernel,
        out_type=[rows_t, rows_t, rows_t],
        mesh=mesh,
        scratch_types=(
            [pltpu.VMEM((b_per_w,), jnp.int32) for _ in range(3)]
            + [pltpu.VMEM((_CHUNK, _EMBED), jnp.float32) for _ in range(_NBUF)]
            + [pltpu.SemaphoreType.DMA for _ in range(2 * _NBUF + 1)]
        ),
    )
    def gather_kernel(ss_hbm, ps_hbm, os_hbm, ent_hbm, rel_hbm,
                      s_out, p_out, o_out, sidx, pidx, oidx, *rest):
        bufs = rest[:_NBUF]
        gsem = rest[_NBUF:2 * _NBUF]
        wsem = rest[2 * _NBUF:3 * _NBUF]
        isem = rest[3 * _NBUF]
        wid = lax.axis_index("s") * 2 + lax.axis_index("c")
        base = wid * b_per_w
        gbase = offset + base

        ih = [pltpu.async_copy(src.at[pl.ds(gbase, b_per_w)], dst, isem)
              for src, dst in ((ss_hbm, sidx), (ps_hbm, pidx), (os_hbm, oidx))]
        for h in ih:
            h.wait()

        items = []
        for idxr, tab, out in ((sidx, ent_hbm, s_out),
                               (pidx, rel_hbm, p_out),
                               (oidx, ent_hbm, o_out)):
            for c in range(0, b_per_w, _CHUNK):
                items.append((idxr, c, tab, out))

        g_h = [None] * _NBUF
        w_h = [None] * _NBUF
        prev = None

        def _start_writeback(i, j):
            _, off, _, out = items[i]
            g_h[j].wait()
            w_h[j] = pltpu.async_copy(
                bufs[j], out.at[pl.ds(base + off, _CHUNK)], wsem[j])

        for i, (idxr, off, tab, _) in enumerate(items):
            j = i % _NBUF
            if w_h[j] is not None:
                w_h[j].wait()
                w_h[j] = None
            g_h[j] = pltpu.async_copy(
                tab.at[idxr.at[pl.ds(off, _CHUNK)]], bufs[j], gsem[j])
            if prev is not None:
                _start_writeback(*prev)
            prev = (i, j)
        _start_writeback(*prev)
        for j in range(_NBUF):
            if w_h[j] is not None:
                w_h[j].wait()

    return gather_kernel(ss, ps, os_idx, ent, rel)


def _score_block(s_ref, p_ref, o_ref, out_ref):
    def _norm(x):
        n = jnp.sqrt(jnp.sum(x * x, axis=-1, keepdims=True))
        return x / jnp.maximum(n, 1e-12)

    d = (_norm(o_ref[...]) - _norm(s_ref[...])) - _norm(p_ref[...]) + 1e-6
    out_ref[...] = jnp.sqrt(jnp.sum(d * d, axis=-1))


def _tc_score(s_rows, p_rows, o_rows):
    batch = s_rows.shape[0]
    blk = 2048
    row_spec = pl.BlockSpec((blk, _EMBED), lambda i: (i, 0))
    return pl.pallas_call(
        _score_block,
        grid=(batch // blk,),
        in_specs=[row_spec, row_spec, row_spec],
        out_specs=pl.BlockSpec((blk,), lambda i: (i,)),
        out_shape=jax.ShapeDtypeStruct((batch,), jnp.float32),
    )(s_rows, p_rows, o_rows)


def kernel(ss, ps, os, ent_embedding, rel_embedding):
    ss = ss.astype(jnp.int32)
    ps = ps.astype(jnp.int32)
    os_idx = os.astype(jnp.int32)
    batch = ss.shape[0]
    n_slices = 2
    sl = batch // n_slices
    scores = []
    for k in range(n_slices):
        s_rows, p_rows, o_rows = _sc_gather(
            ss, ps, os_idx, ent_embedding, rel_embedding, k * sl, sl)
        scores.append(_tc_score(s_rows, p_rows, o_rows))
    return jnp.concatenate(scores)


# speedup vs baseline: 1.0745x; 1.0336x over previous
import functools

import jax
import jax.numpy as jnp
from jax import lax
from jax.experimental import pallas as pl
from jax.experimental.pallas import tpu as pltpu
from jax.experimental.pallas import tpu_sc as plsc

_EMBED = 128
_NUM_WORKERS = 32
_CHUNK = 128


_NBUF = 4


def _sc_gather(ss, ps, os_idx, ent, rel, offset, length):
    b_per_w = length // _NUM_WORKERS
    rows_t = jax.ShapeDtypeStruct((length, _EMBED), jnp.float32)
    mesh = plsc.VectorSubcoreMesh(core_axis_name="c", subcore_axis_name="s")

    @functools.partial(
        pl.kernel,
        out_type=[rows_t, rows_t, rows_t],
        mesh=mesh,
        scratch_types=(
            [pltpu.VMEM((b_per_w,), jnp.int32) for _ in range(3)]
            + [pltpu.VMEM((_CHUNK, _EMBED), jnp.float32) for _ in range(_NBUF)]
            + [pltpu.SemaphoreType.DMA for _ in range(2 * _NBUF + 1)]
        ),
    )
    def gather_kernel(ss_hbm, ps_hbm, os_hbm, ent_hbm, rel_hbm,
                      s_out, p_out, o_out, sidx, pidx, oidx, *rest):
        bufs = rest[:_NBUF]
        gsem = rest[_NBUF:2 * _NBUF]
        wsem = rest[2 * _NBUF:3 * _NBUF]
        isem = rest[3 * _NBUF]
        wid = lax.axis_index("s") * 2 + lax.axis_index("c")
        base = wid * b_per_w
        gbase = offset + base

        ih = [pltpu.async_copy(src.at[pl.ds(gbase, b_per_w)], dst, isem)
              for src, dst in ((ss_hbm, sidx), (ps_hbm, pidx), (os_hbm, oidx))]
        for h in ih:
            h.wait()

        items = []
        for idxr, tab, out in ((sidx, ent_hbm, s_out),
                               (pidx, rel_hbm, p_out),
                               (oidx, ent_hbm, o_out)):
            for c in range(0, b_per_w, _CHUNK):
                items.append((idxr, c, tab, out))

        g_h = [None] * _NBUF
        w_h = [None] * _NBUF
        prev = None

        def _start_writeback(i, j):
            _, off, _, out = items[i]
            g_h[j].wait()
            w_h[j] = pltpu.async_copy(
                bufs[j], out.at[pl.ds(base + off, _CHUNK)], wsem[j])

        for i, (idxr, off, tab, _) in enumerate(items):
            j = i % _NBUF
            if w_h[j] is not None:
                w_h[j].wait()
                w_h[j] = None
            g_h[j] = pltpu.async_copy(
                tab.at[idxr.at[pl.ds(off, _CHUNK)]], bufs[j], gsem[j])
            if prev is not None:
                _start_writeback(*prev)
            prev = (i, j)
        _start_writeback(*prev)
        for j in range(_NBUF):
            if w_h[j] is not None:
                w_h[j].wait()

    return gather_kernel(ss, ps, os_idx, ent, rel)


def _score_block(s_ref, p_ref, o_ref, out_ref):
    def _norm(x):
        sq = jnp.sum(x * x, axis=-1, keepdims=True)
        return x * jax.lax.rsqrt(jnp.maximum(sq, 1e-24))

    d = (_norm(o_ref[...]) - _norm(s_ref[...])) - _norm(p_ref[...]) + 1e-6
    out_ref[...] = jnp.sqrt(jnp.sum(d * d, axis=-1))


def _tc_score(s_rows, p_rows, o_rows):
    batch = s_rows.shape[0]
    blk = 2048
    row_spec = pl.BlockSpec((blk, _EMBED), lambda i: (i, 0))
    return pl.pallas_call(
        _score_block,
        grid=(batch // blk,),
        in_specs=[row_spec, row_spec, row_spec],
        out_specs=pl.BlockSpec((blk,), lambda i: (i,)),
        out_shape=jax.ShapeDtypeStruct((batch,), jnp.float32),
    )(s_rows, p_rows, o_rows)


def kernel(ss, ps, os, ent_embedding, rel_embedding):
    ss = ss.astype(jnp.int32)
    ps = ps.astype(jnp.int32)
    os_idx = os.astype(jnp.int32)
    batch = ss.shape[0]
    n_slices = 2
    sl = batch // n_slices
    scores = []
    for k in range(n_slices):
        s_rows, p_rows, o_rows = _sc_gather(
            ss, ps, os_idx, ent_embedding, rel_embedding, k * sl, sl)
        scores.append(_tc_score(s_rows, p_rows, o_rows))
    return jnp.concatenate(scores)


# speedup vs baseline: 1.1782x; 1.0965x over previous
import functools

import jax
import jax.numpy as jnp
from jax import lax
from jax.experimental import pallas as pl
from jax.experimental.pallas import tpu as pltpu
from jax.experimental.pallas import tpu_sc as plsc

_EMBED = 128
_NUM_WORKERS = 32
_CHUNK = 128
_NBUF = 6
_LAG = 4


def _sc_gather(ss, ps, os_idx, ent, rel, offset, length):
    b_per_w = length // _NUM_WORKERS
    rows_t = jax.ShapeDtypeStruct((length, _EMBED), jnp.float32)
    mesh = plsc.VectorSubcoreMesh(core_axis_name="c", subcore_axis_name="s")

    @functools.partial(
        pl.kernel,
        out_type=[rows_t, rows_t, rows_t],
        mesh=mesh,
        scratch_types=(
            [pltpu.VMEM((b_per_w,), jnp.int32) for _ in range(3)]
            + [pltpu.VMEM((_CHUNK, _EMBED), jnp.float32) for _ in range(_NBUF)]
            + [pltpu.SemaphoreType.DMA for _ in range(_NBUF + 3)]
        ),
    )
    def gather_kernel(ss_hbm, ps_hbm, os_hbm, ent_hbm, rel_hbm,
                      s_out, p_out, o_out, sidx, pidx, oidx, *rest):
        bufs = rest[:_NBUF]
        gsem = rest[_NBUF:2 * _NBUF]
        isem = rest[2 * _NBUF:2 * _NBUF + 3]
        wid = lax.axis_index("s") * 2 + lax.axis_index("c")
        base = wid * b_per_w
        gbase = offset + base

        idx_handles = {}
        for k, (src, dst) in enumerate(((ss_hbm, sidx), (ps_hbm, pidx),
                                        (os_hbm, oidx))):
            idx_handles[id(dst)] = pltpu.async_copy(
                src.at[pl.ds(gbase, b_per_w)], dst, isem[k])

        items = []
        for idxr, tab, out in ((sidx, ent_hbm, s_out),
                               (pidx, rel_hbm, p_out),
                               (oidx, ent_hbm, o_out)):
            for c in range(0, b_per_w, _CHUNK):
                items.append((idxr, c, tab, out))
        n = len(items)

        g_h = [None] * _NBUF
        w_h = [None] * _NBUF
        for i in range(n + _LAG):
            if i < n:
                idxr, off, tab, _ = items[i]
                j = i % _NBUF
                if w_h[j] is not None:
                    w_h[j].wait()
                    w_h[j] = None
                h = idx_handles.pop(id(idxr), None)
                if h is not None:
                    h.wait()
                g_h[j] = pltpu.async_copy(
                    tab.at[idxr.at[pl.ds(off, _CHUNK)]], bufs[j], gsem[j])
            k = i - _LAG
            if k >= 0:
                jk = k % _NBUF
                g_h[jk].wait()
                _, off_k, _, out_k = items[k]
                w_h[jk] = pltpu.async_copy(
                    bufs[jk], out_k.at[pl.ds(base + off_k, _CHUNK)], gsem[jk])
        for j in range(_NBUF):
            if w_h[j] is not None:
                w_h[j].wait()

    return gather_kernel(ss, ps, os_idx, ent, rel)


def _score_block(s_ref, p_ref, o_ref, out_ref):
    def _norm(x):
        sq = jnp.sum(x * x, axis=-1, keepdims=True)
        return x * jax.lax.rsqrt(jnp.maximum(sq, 1e-24))

    d = (_norm(o_ref[...]) - _norm(s_ref[...])) - _norm(p_ref[...]) + 1e-6
    out_ref[...] = jnp.sqrt(jnp.sum(d * d, axis=-1))


def _tc_score(s_rows, p_rows, o_rows, batch, block_offset, acc=None):
    length = s_rows.shape[0]
    blk = 2048
    grid = length // blk
    row_spec = pl.BlockSpec((blk, _EMBED), lambda i: (i, 0))
    out_spec = pl.BlockSpec((blk,), lambda i: (i + block_offset,))
    kwargs = {}
    operands = [s_rows, p_rows, o_rows]
    if acc is not None:
        in_specs = [row_spec, row_spec, row_spec,
                    pl.BlockSpec(memory_space=pl.ANY)]
        operands.append(acc)
        kwargs["input_output_aliases"] = {3: 0}
    else:
        in_specs = [row_spec, row_spec, row_spec]

    def body(*refs):
        _score_block(refs[0], refs[1], refs[2], refs[-1])

    return pl.pallas_call(
        body,
        grid=(grid,),
        in_specs=in_specs,
        out_specs=out_spec,
        out_shape=jax.ShapeDtypeStruct((batch,), jnp.float32),
        **kwargs,
    )(*operands)


def kernel(ss, ps, os, ent_embedding, rel_embedding):
    ss = ss.astype(jnp.int32)
    ps = ps.astype(jnp.int32)
    os = os.astype(jnp.int32)
    batch = ss.shape[0]
    n_slices = 2
    sl = batch // n_slices
    out = None
    for k in range(n_slices):
        s_rows, p_rows, o_rows = _sc_gather(
            ss, ps, os, ent_embedding, rel_embedding, k * sl, sl)
        out = _tc_score(s_rows, p_rows, o_rows, batch,
                        k * (sl // 2048), out)
    return out
